# Initial kernel scaffold; baseline (speedup 1.0000x reference)
import jax
import jax.numpy as jnp
from jax.experimental import pallas as pl


def kernel(x, edge_index, edge_attr, batch, params):
    raise NotImplementedError("write your pallas kernel here")


# speedup vs baseline: 4.4590x; 4.4590x over previous
import functools

import jax
import jax.numpy as jnp
from jax import lax
from jax.experimental import pallas as pl
from jax.experimental.pallas import tpu as pltpu
from jax.experimental.pallas import tpu_sc as plsc

N = 10000
E = 320000
D = 128
DE = 16
H = 128
G = 64

NC = 2
NS = 16
NW = NC * NS
EPW = E // NW
C = 80
NCH = EPW // C
KG = 5
NP = 10240
RPW = NP // NS


@functools.cache
def _sc_kernels():
    mesh = plsc.VectorSubcoreMesh(core_axis_name="c", subcore_axis_name="s",
                                  num_cores=NC, num_subcores=NS)

    @functools.partial(
        pl.kernel,
        out_type=jax.ShapeDtypeStruct((NC, NP, 16), jnp.float32),
        mesh=mesh,
        compiler_params=pltpu.CompilerParams(needs_layout_passes=False),
        scratch_types=[
            pltpu.VMEM((NCH, C), jnp.int32),
            pltpu.VMEM((C, 16), jnp.float32),
            pltpu.VMEM_SHARED((NP, 16), jnp.float32),
        ],
    )
    def _deg_kernel(row_hbm, zeros_hbm, out_hbm, idx_v, ones_v, acc):
        cid = lax.axis_index("c")
        sid = lax.axis_index("s")
        wid = sid * NC + cid
        pltpu.sync_copy(zeros_hbm.at[pl.ds(sid * RPW, RPW)],
                        acc.at[pl.ds(sid * RPW, RPW)])
        def fill(i, _):
            ones_v[i] = jnp.ones((16,), jnp.float32)
            return 0
        lax.fori_loop(0, C, fill, 0)
        pltpu.sync_copy(row_hbm.at[wid], idx_v)
        plsc.subcore_barrier()
        def step(j, _):
            pltpu.sync_copy(ones_v, acc.at[idx_v.at[j]], add=True)
            return 0
        lax.fori_loop(0, NCH, step, 0)
        plsc.subcore_barrier()
        pltpu.sync_copy(acc.at[pl.ds(sid * RPW, RPW)],
                        out_hbm.at[cid].at[pl.ds(sid * RPW, RPW)])

    @functools.partial(
        pl.kernel,
        out_type=jax.ShapeDtypeStruct((E,), jnp.float32),
        mesh=mesh,
        compiler_params=pltpu.CompilerParams(needs_layout_passes=False),
        scratch_types=[
            pltpu.VMEM((N,), jnp.float32),
            pltpu.VMEM((EPW,), jnp.int32),
            pltpu.VMEM((EPW,), jnp.int32),
            pltpu.VMEM((EPW,), jnp.float32),
        ],
    )
    def _norm_kernel(dis_hbm, row_hbm, col_hbm, out_hbm,
                     dis_v, row_v, col_v, nrm_v):
        wid = lax.axis_index("s") * NC + lax.axis_index("c")
        base = wid * EPW
        pltpu.sync_copy(dis_hbm, dis_v)
        pltpu.sync_copy(row_hbm.at[pl.ds(base, EPW)], row_v)
        pltpu.sync_copy(col_hbm.at[pl.ds(base, EPW)], col_v)
        def step(j, _):
            r = row_v[pl.ds(j * 16, 16)]
            c = col_v[pl.ds(j * 16, 16)]
            dr = plsc.load_gather(dis_v, [r])
            dc = plsc.load_gather(dis_v, [c])
            nrm_v[pl.ds(j * 16, 16)] = dr * dc
            return 0
        lax.fori_loop(0, EPW // 16, step, 0)
        pltpu.sync_copy(nrm_v, out_hbm.at[pl.ds(base, EPW)])

    @functools.partial(
        pl.kernel,
        out_type=jax.ShapeDtypeStruct((E, H), jnp.float32),
        mesh=mesh,
        compiler_params=pltpu.CompilerParams(needs_layout_passes=False),
        scratch_types=[
            pltpu.VMEM((NCH, C), jnp.int32),
            pltpu.VMEM((KG, C, H), jnp.float32),
            pltpu.SemaphoreType.DMA,
        ],
    )
    def _gather_kernel(h_hbm, row_hbm, out_hbm, idx_v, buf, sem):
        wid = lax.axis_index("s") * NC + lax.axis_index("c")
        base = wid * EPW
        pltpu.sync_copy(row_hbm.at[wid], idx_v)
        def group(g, _):
            descs = [
                pltpu.async_copy(h_hbm.at[idx_v.at[g * KG + k]],
                                 buf.at[k], sem)
                for k in range(KG)
            ]
            for k in range(KG):
                descs[k].wait()
            for k in range(KG):
                pltpu.sync_copy(buf.at[k],
                                out_hbm.at[pl.ds(base + (g * KG + k) * C, C)])
            return 0
        lax.fori_loop(0, NCH // KG, group, 0)

    @functools.partial(
        pl.kernel,
        out_type=jax.ShapeDtypeStruct((NC, NP, H), jnp.float32),
        mesh=mesh,
        compiler_params=pltpu.CompilerParams(needs_layout_passes=False),
        scratch_types=[
            pltpu.VMEM((NCH, C), jnp.int32),
            pltpu.VMEM((C, H), jnp.float32),
            pltpu.VMEM_SHARED((NP, H), jnp.float32),
        ],
    )
    def _scatter_kernel(msg_hbm, col_hbm, zeros_hbm, out_hbm,
                        col_v, buf, acc):
        cid = lax.axis_index("c")
        sid = lax.axis_index("s")
        wid = sid * NC + cid
        base = wid * EPW
        pltpu.sync_copy(zeros_hbm.at[pl.ds(sid * RPW, RPW)],
                        acc.at[pl.ds(sid * RPW, RPW)])
        pltpu.sync_copy(col_hbm.at[wid], col_v)
        plsc.subcore_barrier()
        def step(j, _):
            pltpu.sync_copy(msg_hbm.at[pl.ds(base + j * C, C)], buf)
            pltpu.sync_copy(buf, acc.at[col_v.at[j]], add=True)
            return 0
        lax.fori_loop(0, NCH, step, 0)
        plsc.subcore_barrier()
        pltpu.sync_copy(acc.at[pl.ds(sid * RPW, RPW)],
                        out_hbm.at[cid].at[pl.ds(sid * RPW, RPW)])

    return _deg_kernel, _norm_kernel, _gather_kernel, _scatter_kernel


def _dot_t(a, b):
    return lax.dot_general(a, b, (((1,), (1,)), ((), ())),
                           preferred_element_type=jnp.float32)


def _first_layer_body(x_ref, w_ref, b_ref, root_ref, deg_ref, h_ref, self_ref):
    h = _dot_t(x_ref[...], w_ref[...]) + b_ref[...]
    h_ref[...] = h
    self_ref[...] = jnp.maximum(h + root_ref[...], 0.0) / deg_ref[...]


def _node_update_body(p0_ref, p1_ref, self_ref, g_ref, bb_ref, w_ref, b_ref,
                      root_ref, deg_ref, bn_ref, h_ref, selfo_ref):
    a = jnp.maximum(p0_ref[...] + p1_ref[...] + self_ref[...], 0.0)
    mu = jnp.mean(a, axis=0, keepdims=True)
    var = jnp.mean((a - mu) ** 2, axis=0, keepdims=True)
    bn = (a - mu) * lax.rsqrt(var + 1e-5) * g_ref[...] + bb_ref[...]
    bn_ref[...] = bn
    h = _dot_t(bn, w_ref[...]) + b_ref[...]
    h_ref[...] = h
    selfo_ref[...] = jnp.maximum(h + root_ref[...], 0.0) / deg_ref[...]


def _msg_body(ea_ref, hrow_ref, norm_ref, w1_ref, b1_ref, w2_ref, b2_ref,
              out_ref):
    e1 = jnp.maximum(_dot_t(ea_ref[...], w1_ref[...]) + b1_ref[...], 0.0)
    e = _dot_t(e1, w2_ref[...]) + b2_ref[...]
    out_ref[...] = norm_ref[...] * jnp.maximum(e + hrow_ref[...], 0.0)


def _tail_body(p0_ref, p1_ref, self_ref, g_ref, bb_ref,
               bn1_ref, bn2_ref, bn3_ref, batch_ref,
               fc1w_ref, fc1b_ref, fc4w_ref, fc4b_ref, out_ref):
    a = jnp.maximum(p0_ref[...] + p1_ref[...] + self_ref[...], 0.0)
    mu = jnp.mean(a, axis=0, keepdims=True)
    var = jnp.mean((a - mu) ** 2, axis=0, keepdims=True)
    bn4 = (a - mu) * lax.rsqrt(var + 1e-5) * g_ref[...] + bb_ref[...]
    gid = lax.broadcasted_iota(jnp.int32, (G, N), 0)
    oh = (batch_ref[...] == gid).astype(jnp.float32)
    cnt = jnp.sum(oh, axis=1, keepdims=True)
    inv = 1.0 / jnp.maximum(cnt, 1.0)
    pooled = []
    for bn in (bn1_ref[...], bn2_ref[...], bn3_ref[...], bn4):
        s = lax.dot_general(oh, bn, (((1,), (0,)), ((), ())),
                            preferred_element_type=jnp.float32)
        pooled.append(s * inv)
    pooled = jnp.concatenate(pooled, axis=1)
    h = jnp.maximum(_dot_t(pooled, fc1w_ref[...]) + fc1b_ref[...], 0.0)
    out_ref[...] = _dot_t(h, fc4w_ref[...]) + fc4b_ref[...]


_f32 = jnp.float32

_first_layer = pl.pallas_call(
    _first_layer_body,
    out_shape=(jax.ShapeDtypeStruct((N, H), _f32),
               jax.ShapeDtypeStruct((N, H), _f32)),
)

_node_update = pl.pallas_call(
    _node_update_body,
    out_shape=(jax.ShapeDtypeStruct((N, H), _f32),
               jax.ShapeDtypeStruct((N, H), _f32),
               jax.ShapeDtypeStruct((N, H), _f32)),
)

EB = 2560

_msg = pl.pallas_call(
    _msg_body,
    grid=(E // EB,),
    in_specs=[
        pl.BlockSpec((EB, DE), lambda i: (i, 0)),
        pl.BlockSpec((EB, H), lambda i: (i, 0)),
        pl.BlockSpec((EB, 1), lambda i: (i, 0)),
        pl.BlockSpec((H, DE), lambda i: (0, 0)),
        pl.BlockSpec((1, H), lambda i: (0, 0)),
        pl.BlockSpec((H, H), lambda i: (0, 0)),
        pl.BlockSpec((1, H), lambda i: (0, 0)),
    ],
    out_specs=pl.BlockSpec((EB, H), lambda i: (i, 0)),
    out_shape=jax.ShapeDtypeStruct((E, H), _f32),
)

_tail = pl.pallas_call(
    _tail_body,
    out_shape=jax.ShapeDtypeStruct((G, H), _f32),
)


def kernel(x, edge_index, edge_attr, batch, params):
    row = edge_index[0]
    col = edge_index[1]
    row3d = row.reshape(NW, NCH, C)
    col3d = col.reshape(NW, NCH, C)
    z16 = jnp.zeros((NP, 16), _f32)
    z128 = jnp.zeros((NP, H), _f32)
    _deg_kernel, _norm_kernel, _gather_kernel, _scatter_kernel = _sc_kernels()

    degp = _deg_kernel(row3d, z16)
    deg = 1.0 + degp[0, :N, 0] + degp[1, :N, 0]
    dis = lax.rsqrt(deg)
    norm = _norm_kernel(dis, row, col)
    norm2d = norm.reshape(E, 1)
    deg2d = deg.reshape(N, 1)
    batch2d = batch.reshape(1, N)

    p1p = params['conv1']
    h, selfv = _first_layer(x, p1p['lin_w'], p1p['lin_b'].reshape(1, H),
                            p1p['root'], deg2d)
    bns = []
    for l in range(1, 5):
        cp = params[f'conv{l}']
        hrow = _gather_kernel(h, row3d)
        msg = _msg(edge_attr, hrow, norm2d,
                   cp['be1_w'], cp['be1_b'].reshape(1, H),
                   cp['be2_w'], cp['be2_b'].reshape(1, H))
        p = _scatter_kernel(msg, col3d, z128)
        g = params[f'bn_g{l}'].reshape(1, H)
        bb = params[f'bn_b{l}'].reshape(1, H)
        if l < 4:
            np_ = params[f'conv{l + 1}']
            bn, h, selfv = _node_update(p[0, :N], p[1, :N], selfv, g, bb,
                                        np_['lin_w'],
                                        np_['lin_b'].reshape(1, H),
                                        np_['root'], deg2d)
            bns.append(bn)
        else:
            out = _tail(p[0, :N], p[1, :N], selfv, g, bb,
                        bns[0], bns[1], bns[2], batch2d,
                        params['fc1_w'], params['fc1_b'].reshape(1, H),
                        params['fc4_w'], params['fc4_b'].reshape(1, H))
    return out


# speedup vs baseline: 4.5289x; 1.0157x over previous
import functools

import jax
import jax.numpy as jnp
from jax import lax
from jax.experimental import pallas as pl
from jax.experimental.pallas import tpu as pltpu
from jax.experimental.pallas import tpu_sc as plsc

N = 10000
E = 320000
D = 128
DE = 16
H = 128
G = 64

NC = 2
NS = 16
NW = NC * NS
EPW = E // NW
C = 80
NCH = EPW // C
KG = 5
NP = 10240
RPW = NP // NS


@functools.cache
def _sc_kernels():
    mesh = plsc.VectorSubcoreMesh(core_axis_name="c", subcore_axis_name="s",
                                  num_cores=NC, num_subcores=NS)

    @functools.partial(
        pl.kernel,
        out_type=jax.ShapeDtypeStruct((NC, NP, 16), jnp.float32),
        mesh=mesh,
        compiler_params=pltpu.CompilerParams(needs_layout_passes=False),
        scratch_types=[
            pltpu.VMEM((NCH, C), jnp.int32),
            pltpu.VMEM((C, 16), jnp.float32),
            pltpu.VMEM_SHARED((NP, 16), jnp.float32),
        ],
    )
    def _deg_kernel(row_hbm, zeros_hbm, out_hbm, idx_v, ones_v, acc):
        cid = lax.axis_index("c")
        sid = lax.axis_index("s")
        wid = sid * NC + cid
        pltpu.sync_copy(zeros_hbm.at[pl.ds(sid * RPW, RPW)],
                        acc.at[pl.ds(sid * RPW, RPW)])
        def fill(i, _):
            ones_v[i] = jnp.ones((16,), jnp.float32)
            return 0
        lax.fori_loop(0, C, fill, 0)
        pltpu.sync_copy(row_hbm.at[wid], idx_v)
        plsc.subcore_barrier()
        def step(j, _):
            pltpu.sync_copy(ones_v, acc.at[idx_v.at[j]], add=True)
            return 0
        lax.fori_loop(0, NCH, step, 0)
        plsc.subcore_barrier()
        pltpu.sync_copy(acc.at[pl.ds(sid * RPW, RPW)],
                        out_hbm.at[cid].at[pl.ds(sid * RPW, RPW)])

    @functools.partial(
        pl.kernel,
        out_type=jax.ShapeDtypeStruct((E,), jnp.float32),
        mesh=mesh,
        compiler_params=pltpu.CompilerParams(needs_layout_passes=False),
        scratch_types=[
            pltpu.VMEM((N,), jnp.float32),
            pltpu.VMEM((EPW,), jnp.int32),
            pltpu.VMEM((EPW,), jnp.int32),
            pltpu.VMEM((EPW,), jnp.float32),
        ],
    )
    def _norm_kernel(dis_hbm, row_hbm, col_hbm, out_hbm,
                     dis_v, row_v, col_v, nrm_v):
        wid = lax.axis_index("s") * NC + lax.axis_index("c")
        base = wid * EPW
        pltpu.sync_copy(dis_hbm, dis_v)
        pltpu.sync_copy(row_hbm.at[pl.ds(base, EPW)], row_v)
        pltpu.sync_copy(col_hbm.at[pl.ds(base, EPW)], col_v)
        def step(j, _):
            r = row_v[pl.ds(j * 16, 16)]
            c = col_v[pl.ds(j * 16, 16)]
            dr = plsc.load_gather(dis_v, [r])
            dc = plsc.load_gather(dis_v, [c])
            nrm_v[pl.ds(j * 16, 16)] = dr * dc
            return 0
        lax.fori_loop(0, EPW // 16, step, 0)
        pltpu.sync_copy(nrm_v, out_hbm.at[pl.ds(base, EPW)])

    @functools.partial(
        pl.kernel,
        out_type=jax.ShapeDtypeStruct((E, H), jnp.float32),
        mesh=mesh,
        compiler_params=pltpu.CompilerParams(needs_layout_passes=False),
        scratch_types=[
            pltpu.VMEM((NCH, C), jnp.int32),
            pltpu.VMEM((2 * KG, C, H), jnp.float32),
            pltpu.SemaphoreType.DMA,
            pltpu.SemaphoreType.DMA,
            pltpu.SemaphoreType.DMA,
        ],
    )
    def _gather_kernel(h_hbm, row_hbm, out_hbm, idx_v, buf, gsem,
                       wsem0, wsem1):
        wid = lax.axis_index("s") * NC + lax.axis_index("c")
        base = wid * EPW
        pltpu.sync_copy(row_hbm.at[wid], idx_v)

        def do_group(gg, p5, wsem):
            @pl.when(gg >= 2)
            def _():
                for k in range(KG):
                    pltpu.make_async_copy(
                        buf.at[p5 + k],
                        out_hbm.at[pl.ds(base + (gg * KG + k) * C, C)],
                        wsem).wait()
            descs = [
                pltpu.async_copy(h_hbm.at[idx_v.at[gg * KG + k]],
                                 buf.at[p5 + k], gsem)
                for k in range(KG)
            ]
            for k in range(KG):
                descs[k].wait()
            for k in range(KG):
                pltpu.async_copy(
                    buf.at[p5 + k],
                    out_hbm.at[pl.ds(base + (gg * KG + k) * C, C)], wsem)

        def pair(t, _):
            do_group(2 * t, 0, wsem0)
            do_group(2 * t + 1, KG, wsem1)
            return 0
        lax.fori_loop(0, (NCH // KG) // 2, pair, 0)
        do_group((NCH // KG) - 1, 0, wsem0)
        for k in range(KG):
            pltpu.make_async_copy(buf.at[k], out_hbm.at[pl.ds(base, C)],
                                  wsem0).wait()
            pltpu.make_async_copy(buf.at[KG + k], out_hbm.at[pl.ds(base, C)],
                                  wsem1).wait()

    @functools.partial(
        pl.kernel,
        out_type=jax.ShapeDtypeStruct((NC, NP, H), jnp.float32),
        mesh=mesh,
        compiler_params=pltpu.CompilerParams(needs_layout_passes=False),
        scratch_types=[
            pltpu.VMEM((NCH, C), jnp.int32),
            pltpu.VMEM((C, H), jnp.float32),
            pltpu.VMEM_SHARED((NP, H), jnp.float32),
        ],
    )
    def _scatter_kernel(msg_hbm, col_hbm, zeros_hbm, out_hbm,
                        col_v, buf, acc):
        cid = lax.axis_index("c")
        sid = lax.axis_index("s")
        wid = sid * NC + cid
        base = wid * EPW
        pltpu.sync_copy(zeros_hbm.at[pl.ds(sid * RPW, RPW)],
                        acc.at[pl.ds(sid * RPW, RPW)])
        pltpu.sync_copy(col_hbm.at[wid], col_v)
        plsc.subcore_barrier()
        def step(j, _):
            pltpu.sync_copy(msg_hbm.at[pl.ds(base + j * C, C)], buf)
            pltpu.sync_copy(buf, acc.at[col_v.at[j]], add=True)
            return 0
        lax.fori_loop(0, NCH, step, 0)
        plsc.subcore_barrier()
        pltpu.sync_copy(acc.at[pl.ds(sid * RPW, RPW)],
                        out_hbm.at[cid].at[pl.ds(sid * RPW, RPW)])

    return _deg_kernel, _norm_kernel, _gather_kernel, _scatter_kernel


def _dot_t(a, b):
    return lax.dot_general(a, b, (((1,), (1,)), ((), ())),
                           preferred_element_type=jnp.float32)


def _first_layer_body(x_ref, w_ref, b_ref, root_ref, deg_ref, h_ref, self_ref):
    h = _dot_t(x_ref[...], w_ref[...]) + b_ref[...]
    h_ref[...] = h
    self_ref[...] = jnp.maximum(h + root_ref[...], 0.0) / deg_ref[...]


def _node_update_body(p0_ref, p1_ref, self_ref, g_ref, bb_ref, w_ref, b_ref,
                      root_ref, deg_ref, bn_ref, h_ref, selfo_ref):
    a = jnp.maximum(p0_ref[...] + p1_ref[...] + self_ref[...], 0.0)
    mu = jnp.mean(a, axis=0, keepdims=True)
    var = jnp.mean((a - mu) ** 2, axis=0, keepdims=True)
    bn = (a - mu) * lax.rsqrt(var + 1e-5) * g_ref[...] + bb_ref[...]
    bn_ref[...] = bn
    h = _dot_t(bn, w_ref[...]) + b_ref[...]
    h_ref[...] = h
    selfo_ref[...] = jnp.maximum(h + root_ref[...], 0.0) / deg_ref[...]


def _msg_body(ea_ref, hrow_ref, norm_ref, w1_ref, b1_ref, w2_ref, b2_ref,
              out_ref):
    e1 = jnp.maximum(_dot_t(ea_ref[...], w1_ref[...]) + b1_ref[...], 0.0)
    e = _dot_t(e1, w2_ref[...]) + b2_ref[...]
    out_ref[...] = norm_ref[...] * jnp.maximum(e + hrow_ref[...], 0.0)


def _tail_body(p0_ref, p1_ref, self_ref, g_ref, bb_ref,
               bn1_ref, bn2_ref, bn3_ref, batch_ref,
               fc1w_ref, fc1b_ref, fc4w_ref, fc4b_ref, out_ref):
    a = jnp.maximum(p0_ref[...] + p1_ref[...] + self_ref[...], 0.0)
    mu = jnp.mean(a, axis=0, keepdims=True)
    var = jnp.mean((a - mu) ** 2, axis=0, keepdims=True)
    bn4 = (a - mu) * lax.rsqrt(var + 1e-5) * g_ref[...] + bb_ref[...]
    gid = lax.broadcasted_iota(jnp.int32, (G, N), 0)
    oh = (batch_ref[...] == gid).astype(jnp.float32)
    cnt = jnp.sum(oh, axis=1, keepdims=True)
    inv = 1.0 / jnp.maximum(cnt, 1.0)
    pooled = []
    for bn in (bn1_ref[...], bn2_ref[...], bn3_ref[...], bn4):
        s = lax.dot_general(oh, bn, (((1,), (0,)), ((), ())),
                            preferred_element_type=jnp.float32)
        pooled.append(s * inv)
    pooled = jnp.concatenate(pooled, axis=1)
    h = jnp.maximum(_dot_t(pooled, fc1w_ref[...]) + fc1b_ref[...], 0.0)
    out_ref[...] = _dot_t(h, fc4w_ref[...]) + fc4b_ref[...]


_f32 = jnp.float32

_first_layer = pl.pallas_call(
    _first_layer_body,
    out_shape=(jax.ShapeDtypeStruct((N, H), _f32),
               jax.ShapeDtypeStruct((N, H), _f32)),
)

_node_update = pl.pallas_call(
    _node_update_body,
    out_shape=(jax.ShapeDtypeStruct((N, H), _f32),
               jax.ShapeDtypeStruct((N, H), _f32),
               jax.ShapeDtypeStruct((N, H), _f32)),
)

EB = 2560

_msg = pl.pallas_call(
    _msg_body,
    grid=(E // EB,),
    in_specs=[
        pl.BlockSpec((EB, DE), lambda i: (i, 0)),
        pl.BlockSpec((EB, H), lambda i: (i, 0)),
        pl.BlockSpec((EB, 1), lambda i: (i, 0)),
        pl.BlockSpec((H, DE), lambda i: (0, 0)),
        pl.BlockSpec((1, H), lambda i: (0, 0)),
        pl.BlockSpec((H, H), lambda i: (0, 0)),
        pl.BlockSpec((1, H), lambda i: (0, 0)),
    ],
    out_specs=pl.BlockSpec((EB, H), lambda i: (i, 0)),
    out_shape=jax.ShapeDtypeStruct((E, H), _f32),
)

_tail = pl.pallas_call(
    _tail_body,
    out_shape=jax.ShapeDtypeStruct((G, H), _f32),
)


def kernel(x, edge_index, edge_attr, batch, params):
    row = edge_index[0]
    col = edge_index[1]
    row3d = row.reshape(NW, NCH, C)
    col3d = col.reshape(NW, NCH, C)
    z16 = jnp.zeros((NP, 16), _f32)
    z128 = jnp.zeros((NP, H), _f32)
    _deg_kernel, _norm_kernel, _gather_kernel, _scatter_kernel = _sc_kernels()

    degp = _deg_kernel(row3d, z16)
    deg = 1.0 + degp[0, :N, 0] + degp[1, :N, 0]
    dis = lax.rsqrt(deg)
    norm = _norm_kernel(dis, row, col)
    norm2d = norm.reshape(E, 1)
    deg2d = deg.reshape(N, 1)
    batch2d = batch.reshape(1, N)

    p1p = params['conv1']
    h, selfv = _first_layer(x, p1p['lin_w'], p1p['lin_b'].reshape(1, H),
                            p1p['root'], deg2d)
    bns = []
    for l in range(1, 5):
        cp = params[f'conv{l}']
        hrow = _gather_kernel(h, row3d)
        msg = _msg(edge_attr, hrow, norm2d,
                   cp['be1_w'], cp['be1_b'].reshape(1, H),
                   cp['be2_w'], cp['be2_b'].reshape(1, H))
        p = _scatter_kernel(msg, col3d, z128)
        g = params[f'bn_g{l}'].reshape(1, H)
        bb = params[f'bn_b{l}'].reshape(1, H)
        if l < 4:
            np_ = params[f'conv{l + 1}']
            bn, h, selfv = _node_update(p[0, :N], p[1, :N], selfv, g, bb,
                                        np_['lin_w'],
                                        np_['lin_b'].reshape(1, H),
                                        np_['root'], deg2d)
            bns.append(bn)
        else:
            out = _tail(p[0, :N], p[1, :N], selfv, g, bb,
                        bns[0], bns[1], bns[2], batch2d,
                        params['fc1_w'], params['fc1_b'].reshape(1, H),
                        params['fc4_w'], params['fc4_b'].reshape(1, H))
    return out
